# baseline (speedup 1.0000x reference)
import jax
import jax.numpy as jnp
from jax.experimental import pallas as pl
from jax.experimental.pallas import tpu as pltpu


def kernel(x, row_emb, col_emb):
    b = x.shape[0]
    h, w = x.shape[-2], x.shape[-1]
    d = row_emb.shape[1]

    def body(col_ref, row_ref, out_ref):
        colT = col_ref[:w, :].T
        rowT = row_ref[:h, :].T
        out_ref[0, :d] = jnp.broadcast_to(colT[:, None, :], (d, h, w))
        out_ref[0, d:] = jnp.broadcast_to(rowT[:, :, None], (d, h, w))

    out = pl.pallas_call(
        body,
        grid=(b,),
        in_specs=[
            pl.BlockSpec(col_emb.shape, lambda i: (0, 0)),
            pl.BlockSpec(row_emb.shape, lambda i: (0, 0)),
        ],
        out_specs=pl.BlockSpec((1, 2 * d, h, w), lambda i: (i, 0, 0, 0)),
        out_shape=jax.ShapeDtypeStruct((b, 2 * d, h, w), jnp.float32),
    )(col_emb, row_emb)
    return out


# speedup vs baseline: 2.9504x; 2.9504x over previous
import jax
import jax.numpy as jnp
from jax.experimental import pallas as pl
from jax.experimental.pallas import tpu as pltpu


def kernel(x, row_emb, col_emb):
    b = x.shape[0]
    h, w = x.shape[-2], x.shape[-1]
    d = row_emb.shape[1]

    def body(col_ref, row_ref, out_ref, scratch, sem):
        colT = col_ref[:w, :].T
        rowT = row_ref[:h, :].T
        for j in range(h):
            scratch[0:d, j * w:(j + 1) * w] = colT
            scratch[d:2 * d, j * w:(j + 1) * w] = jnp.broadcast_to(
                rowT[:, j][:, None], (d, w))
        copies = [
            pltpu.make_async_copy(scratch, out_ref.at[i], sem)
            for i in range(b)
        ]
        for c in copies:
            c.start()
        for c in copies:
            c.wait()

    out = pl.pallas_call(
        body,
        out_specs=pl.BlockSpec(memory_space=pl.ANY),
        out_shape=jax.ShapeDtypeStruct((b, 2 * d, h * w), jnp.float32),
        scratch_shapes=[
            pltpu.VMEM((2 * d, h * w), jnp.float32),
            pltpu.SemaphoreType.DMA,
        ],
    )(col_emb, row_emb)
    return out.reshape(b, 2 * d, h, w)


# speedup vs baseline: 2.9569x; 1.0022x over previous
import jax
import jax.numpy as jnp
from jax.experimental import pallas as pl
from jax.experimental.pallas import tpu as pltpu

_BPG = 2


def kernel(x, row_emb, col_emb):
    b = x.shape[0]
    h, w = x.shape[-2], x.shape[-1]
    d = row_emb.shape[1]
    hw = h * w

    def body(col_ref, row_ref, out_ref, scratch):
        i = pl.program_id(0)

        @pl.when(i == 0)
        def _build():
            colT = col_ref[:w, :].T
            rowT = row_ref[:h, :].T
            for j in range(h):
                scratch[0:d, j * w:(j + 1) * w] = colT
                scratch[d:2 * d, j * w:(j + 1) * w] = jnp.broadcast_to(
                    rowT[:, j][:, None], (d, w))

        out_ref[...] = jnp.broadcast_to(scratch[...][None], (_BPG, 2 * d, hw))

    out = pl.pallas_call(
        body,
        grid=(b // _BPG,),
        in_specs=[
            pl.BlockSpec(col_emb.shape, lambda i: (0, 0)),
            pl.BlockSpec(row_emb.shape, lambda i: (0, 0)),
        ],
        out_specs=pl.BlockSpec((_BPG, 2 * d, hw), lambda i: (i, 0, 0)),
        out_shape=jax.ShapeDtypeStruct((b, 2 * d, hw), jnp.float32),
        scratch_shapes=[pltpu.VMEM((2 * d, hw), jnp.float32)],
    )(col_emb, row_emb)
    return out.reshape(b, 2 * d, h, w)
